# Initial kernel scaffold; baseline (speedup 1.0000x reference)
import jax
import jax.numpy as jnp
from jax.experimental import pallas as pl


def kernel(x, edge_index, W, b):
    raise NotImplementedError("write your pallas kernel here")


# speedup vs baseline: 28.2417x; 28.2417x over previous
import functools

import jax
import jax.numpy as jnp
from jax import lax
from jax.experimental import pallas as pl
from jax.experimental.pallas import tpu as pltpu
from jax.experimental.pallas import tpu_sc as plsc

N_NODES = 10000
N_EDGES = 320000
D = 128

NC = 2
NS = 16
NW = NC * NS
EPT = N_EDGES // NW
CHUNK = 80
NCHUNK = EPT // CHUNK
BASE_ROWS = 624
ZROWS = 208

ROWBLK = 1000
GRID = N_NODES // ROWBLK

_mesh = plsc.VectorSubcoreMesh(core_axis_name="c", subcore_axis_name="s")


@functools.partial(
    pl.kernel,
    out_type=jax.ShapeDtypeStruct((NW, N_NODES), jnp.float32),
    mesh=_mesh,
    compiler_params=pltpu.CompilerParams(needs_layout_passes=False),
    scratch_types=[
        pltpu.VMEM((EPT,), jnp.int32),
        pltpu.VMEM((N_NODES,), jnp.float32),
    ],
)
def _deg_kernel(dst_hbm, out_hbm, dst_v, deg_v):
    c = lax.axis_index("c")
    s = lax.axis_index("s")
    wid = c * NS + s

    zeros16 = jnp.zeros((16,), jnp.float32)

    def zero_body(i, carry):
        deg_v[pl.ds(i * 16, 16)] = zeros16
        return carry

    lax.fori_loop(0, N_NODES // 16, zero_body, 0)

    pltpu.sync_copy(dst_hbm.at[wid], dst_v)

    ones16 = jnp.full((16,), 1.0, jnp.float32)

    def body(i, carry):
        idx = dst_v[pl.ds(i * 16, 16)]
        plsc.addupdate_scatter(deg_v, [idx], ones16)
        return carry

    lax.fori_loop(0, EPT // 16, body, 0)

    pltpu.sync_copy(deg_v, out_hbm.at[wid])


@functools.partial(
    pl.kernel,
    out_type=jax.ShapeDtypeStruct((NC, N_NODES, D), jnp.float32),
    mesh=_mesh,
    compiler_params=pltpu.CompilerParams(needs_layout_passes=False),
    scratch_types=[
        pltpu.VMEM((NCHUNK, CHUNK), jnp.int32),
        pltpu.VMEM((NCHUNK, CHUNK), jnp.int32),
        pltpu.VMEM((CHUNK, D), jnp.float32),
        pltpu.VMEM_SHARED((N_NODES, D), jnp.float32),
        pltpu.SemaphoreType.DMA,
    ],
)
def _edge_kernel(src_hbm, dst_hbm, y_hbm, out_hbm,
                 src_v, dst_v, rows_v, accum, sem):
    c = lax.axis_index("c")
    s = lax.axis_index("s")
    wid = c * NS + s

    zeros16 = jnp.zeros((16,), jnp.float32)

    def zb(i, carry):
        rows_v[i // 8, pl.ds((i % 8) * 16, 16)] = zeros16
        return carry

    lax.fori_loop(0, CHUNK * 8, zb, 0)

    start = BASE_ROWS * s + 8 * jnp.minimum(s, 2)
    for j in range(BASE_ROWS // CHUNK):
        off = pl.multiple_of(start + j * CHUNK, 8)
        pltpu.sync_copy(rows_v, accum.at[pl.ds(off, CHUNK)])
    off64 = pl.multiple_of(start + (BASE_ROWS // CHUNK) * CHUNK, 8)
    pltpu.sync_copy(rows_v.at[pl.ds(0, BASE_ROWS % CHUNK)],
                    accum.at[pl.ds(off64, BASE_ROWS % CHUNK)])

    @pl.when(s < 2)
    def _zero_extra():
        off = pl.multiple_of(start + BASE_ROWS, 8)
        pltpu.sync_copy(rows_v.at[pl.ds(0, 8)], accum.at[pl.ds(off, 8)])

    plsc.subcore_barrier()

    pltpu.sync_copy(src_hbm.at[wid], src_v)
    pltpu.sync_copy(dst_hbm.at[wid], dst_v)

    def body(k, carry):
        pltpu.async_copy(y_hbm.at[src_v.at[k]], rows_v, sem).wait()
        pltpu.sync_copy(rows_v, accum.at[dst_v.at[k]], add=True)
        return carry

    lax.fori_loop(0, NCHUNK, body, 0)
    plsc.subcore_barrier()

    off0 = pl.multiple_of(start, 8)
    pltpu.sync_copy(accum.at[pl.ds(off0, BASE_ROWS)],
                    out_hbm.at[c, pl.ds(off0, BASE_ROWS)])

    @pl.when(s < 2)
    def _write_extra():
        off = pl.multiple_of(start + BASE_ROWS, 8)
        pltpu.sync_copy(accum.at[pl.ds(off, 8)],
                        out_hbm.at[c, pl.ds(off, 8)])


def _mm_body(x_ref, w_ref, degp_ref, y_ref):
    deg = jnp.sum(degp_ref[...], axis=1) + 1.0
    dis = lax.rsqrt(deg)
    xw = jnp.dot(x_ref[...], w_ref[...], preferred_element_type=jnp.float32)
    y_ref[...] = xw * dis[:, None]


def _mm_call(x, W, degp):
    return pl.pallas_call(
        _mm_body,
        grid=(GRID,),
        in_specs=[
            pl.BlockSpec((ROWBLK, D), lambda i: (i, 0)),
            pl.BlockSpec((D, D), lambda i: (0, 0)),
            pl.BlockSpec((ROWBLK, NW), lambda i: (i, 0)),
        ],
        out_specs=pl.BlockSpec((ROWBLK, D), lambda i: (i, 0)),
        out_shape=jax.ShapeDtypeStruct((N_NODES, D), jnp.float32),
    )(x, W, degp)


def _fin_body(q_ref, y_ref, degp_ref, b_ref, o_ref):
    deg = jnp.sum(degp_ref[...], axis=1) + 1.0
    dis = lax.rsqrt(deg)
    t = q_ref[0] + q_ref[1] + y_ref[...]
    o_ref[...] = jnp.maximum(t * dis[:, None] + b_ref[...], 0.0)


def _fin_call(q, y, degp, b2):
    return pl.pallas_call(
        _fin_body,
        grid=(GRID,),
        in_specs=[
            pl.BlockSpec((NC, ROWBLK, D), lambda i: (0, i, 0)),
            pl.BlockSpec((ROWBLK, D), lambda i: (i, 0)),
            pl.BlockSpec((ROWBLK, NW), lambda i: (i, 0)),
            pl.BlockSpec((1, D), lambda i: (0, 0)),
        ],
        out_specs=pl.BlockSpec((ROWBLK, D), lambda i: (i, 0)),
        out_shape=jax.ShapeDtypeStruct((N_NODES, D), jnp.float32),
    )(q, y, degp, b2)


def kernel(x, edge_index, W, b):
    src = edge_index[0].astype(jnp.int32)
    dst = edge_index[1].astype(jnp.int32)
    src3 = src.reshape(NW, NCHUNK, CHUNK)
    dst3 = dst.reshape(NW, NCHUNK, CHUNK)
    dst2 = dst.reshape(NW, EPT)

    degp = _deg_kernel(dst2).T
    y = _mm_call(x, W, degp)
    q = _edge_kernel(src3, dst3, y)
    out = _fin_call(q, y, degp, b.reshape(1, D))
    return out


# speedup vs baseline: 34.7184x; 1.2293x over previous
import functools

import jax
import jax.numpy as jnp
from jax import lax
from jax.experimental import pallas as pl
from jax.experimental.pallas import tpu as pltpu
from jax.experimental.pallas import tpu_sc as plsc

N_NODES = 10000
N_EDGES = 320000
D = 128

NC = 2
NS = 16
NW = NC * NS
EPT = N_EDGES // NW
CHUNK = 80
NCHUNK = EPT // CHUNK
BASE_ROWS = 624
ZROWS = 208

ROWBLK = 1000
GRID = N_NODES // ROWBLK

_mesh = plsc.VectorSubcoreMesh(core_axis_name="c", subcore_axis_name="s")


@functools.partial(
    pl.kernel,
    out_type=jax.ShapeDtypeStruct((NW, N_NODES), jnp.float32),
    mesh=_mesh,
    compiler_params=pltpu.CompilerParams(needs_layout_passes=False),
    scratch_types=[
        pltpu.VMEM((EPT,), jnp.int32),
        pltpu.VMEM((N_NODES,), jnp.float32),
    ],
)
def _deg_kernel(dst_hbm, out_hbm, dst_v, deg_v):
    c = lax.axis_index("c")
    s = lax.axis_index("s")
    wid = c * NS + s

    zeros16 = jnp.zeros((16,), jnp.float32)

    def zero_body(i, carry):
        deg_v[pl.ds(i * 16, 16)] = zeros16
        return carry

    lax.fori_loop(0, N_NODES // 16, zero_body, 0)

    pltpu.sync_copy(dst_hbm.at[wid], dst_v)

    ones16 = jnp.full((16,), 1.0, jnp.float32)

    def body(i, carry):
        idx = dst_v[pl.ds(i * 16, 16)]
        plsc.addupdate_scatter(deg_v, [idx], ones16)
        return carry

    lax.fori_loop(0, EPT // 16, body, 0)

    pltpu.sync_copy(deg_v, out_hbm.at[wid])


@functools.partial(
    pl.kernel,
    out_type=jax.ShapeDtypeStruct((NC, N_NODES, D), jnp.float32),
    mesh=_mesh,
    compiler_params=pltpu.CompilerParams(needs_layout_passes=False),
    scratch_types=[
        pltpu.VMEM((NCHUNK, CHUNK), jnp.int32),
        pltpu.VMEM((2, CHUNK), jnp.int32),
        pltpu.VMEM((2, CHUNK, D), jnp.float32),
        pltpu.VMEM_SHARED((N_NODES, D), jnp.float32),
        pltpu.SemaphoreType.DMA,
        pltpu.SemaphoreType.DMA,
    ],
)
def _edge_kernel(src_hbm, dst_hbm, y_hbm, out_hbm,
                 src_v, dsti, rows_v, accum, sem, isem):
    c = lax.axis_index("c")
    s = lax.axis_index("s")
    wid = c * NS + s

    zeros16 = jnp.zeros((16,), jnp.float32)

    def zb(i, carry):
        rows_v[0, i // 8, pl.ds((i % 8) * 16, 16)] = zeros16
        return carry

    lax.fori_loop(0, CHUNK * 8, zb, 0)

    zrows = rows_v.at[0]
    start = BASE_ROWS * s + 8 * jnp.minimum(s, 2)
    for j in range(BASE_ROWS // CHUNK):
        off = pl.multiple_of(start + j * CHUNK, 8)
        pltpu.sync_copy(zrows, accum.at[pl.ds(off, CHUNK)])
    off64 = pl.multiple_of(start + (BASE_ROWS // CHUNK) * CHUNK, 8)
    pltpu.sync_copy(zrows.at[pl.ds(0, BASE_ROWS % CHUNK)],
                    accum.at[pl.ds(off64, BASE_ROWS % CHUNK)])

    @pl.when(s < 2)
    def _zero_extra():
        off = pl.multiple_of(start + BASE_ROWS, 8)
        pltpu.sync_copy(zrows.at[pl.ds(0, 8)], accum.at[pl.ds(off, 8)])

    plsc.subcore_barrier()

    pltpu.sync_copy(src_hbm.at[wid], src_v)

    pltpu.async_copy(dst_hbm.at[wid, 0], dsti.at[0], isem)
    pltpu.async_copy(y_hbm.at[src_v.at[0]], rows_v.at[0], sem)

    def body(k, carry):
        buf = lax.rem(k, 2)
        pltpu.make_async_copy(y_hbm.at[src_v.at[k]], rows_v.at[buf], sem).wait()
        pltpu.make_async_copy(dst_hbm.at[wid, k], dsti.at[buf], isem).wait()

        @pl.when(k + 1 < NCHUNK)
        def _prefetch():
            pltpu.async_copy(y_hbm.at[src_v.at[k + 1]], rows_v.at[1 - buf], sem)
            pltpu.async_copy(dst_hbm.at[wid, k + 1], dsti.at[1 - buf], isem)

        pltpu.sync_copy(rows_v.at[buf], accum.at[dsti.at[buf]], add=True)
        return carry

    lax.fori_loop(0, NCHUNK, body, 0)
    plsc.subcore_barrier()

    off0 = pl.multiple_of(start, 8)
    pltpu.sync_copy(accum.at[pl.ds(off0, BASE_ROWS)],
                    out_hbm.at[c, pl.ds(off0, BASE_ROWS)])

    @pl.when(s < 2)
    def _write_extra():
        off = pl.multiple_of(start + BASE_ROWS, 8)
        pltpu.sync_copy(accum.at[pl.ds(off, 8)],
                        out_hbm.at[c, pl.ds(off, 8)])


def _mm_body(x_ref, w_ref, degp_ref, y_ref):
    deg = jnp.sum(degp_ref[...], axis=1) + 1.0
    dis = lax.rsqrt(deg)
    xw = jnp.dot(x_ref[...], w_ref[...], preferred_element_type=jnp.float32)
    y_ref[...] = xw * dis[:, None]


def _mm_call(x, W, degp):
    return pl.pallas_call(
        _mm_body,
        grid=(GRID,),
        in_specs=[
            pl.BlockSpec((ROWBLK, D), lambda i: (i, 0)),
            pl.BlockSpec((D, D), lambda i: (0, 0)),
            pl.BlockSpec((ROWBLK, NW), lambda i: (i, 0)),
        ],
        out_specs=pl.BlockSpec((ROWBLK, D), lambda i: (i, 0)),
        out_shape=jax.ShapeDtypeStruct((N_NODES, D), jnp.float32),
    )(x, W, degp)


def _fin_body(q_ref, y_ref, degp_ref, b_ref, o_ref):
    deg = jnp.sum(degp_ref[...], axis=1) + 1.0
    dis = lax.rsqrt(deg)
    t = q_ref[0] + q_ref[1] + y_ref[...]
    o_ref[...] = jnp.maximum(t * dis[:, None] + b_ref[...], 0.0)


def _fin_call(q, y, degp, b2):
    return pl.pallas_call(
        _fin_body,
        grid=(GRID,),
        in_specs=[
            pl.BlockSpec((NC, ROWBLK, D), lambda i: (0, i, 0)),
            pl.BlockSpec((ROWBLK, D), lambda i: (i, 0)),
            pl.BlockSpec((ROWBLK, NW), lambda i: (i, 0)),
            pl.BlockSpec((1, D), lambda i: (0, 0)),
        ],
        out_specs=pl.BlockSpec((ROWBLK, D), lambda i: (i, 0)),
        out_shape=jax.ShapeDtypeStruct((N_NODES, D), jnp.float32),
    )(q, y, degp, b2)


def kernel(x, edge_index, W, b):
    src = edge_index[0].astype(jnp.int32)
    dst = edge_index[1].astype(jnp.int32)
    src3 = src.reshape(NW, NCHUNK, CHUNK)
    dst3 = dst.reshape(NW, NCHUNK, CHUNK)
    dst2 = dst.reshape(NW, EPT)

    degp = _deg_kernel(dst2).T
    y = _mm_call(x, W, degp)
    q = _edge_kernel(src3, dst3, y)
    out = _fin_call(q, y, degp, b.reshape(1, D))
    return out


# speedup vs baseline: 46.8669x; 1.3499x over previous
import functools

import jax
import jax.numpy as jnp
from jax import lax
from jax.experimental import pallas as pl
from jax.experimental.pallas import tpu as pltpu
from jax.experimental.pallas import tpu_sc as plsc

N_NODES = 10000
N_EDGES = 320000
D = 128

NC = 2
NS = 16
NW = NC * NS
EPT = N_EDGES // NW
CHUNK = 80
NCHUNK = EPT // CHUNK
BASE_ROWS = 624
ZROWS = 208

ROWBLK = 1000
GRID = N_NODES // ROWBLK

_mesh = plsc.VectorSubcoreMesh(core_axis_name="c", subcore_axis_name="s")


@functools.partial(
    pl.kernel,
    out_type=jax.ShapeDtypeStruct((NW, N_NODES), jnp.float32),
    mesh=_mesh,
    compiler_params=pltpu.CompilerParams(needs_layout_passes=False),
    scratch_types=[
        pltpu.VMEM((EPT,), jnp.int32),
        pltpu.VMEM((N_NODES,), jnp.float32),
    ],
)
def _deg_kernel(dst_hbm, out_hbm, dst_v, deg_v):
    c = lax.axis_index("c")
    s = lax.axis_index("s")
    wid = c * NS + s

    zeros16 = jnp.zeros((16,), jnp.float32)

    def zero_body(i, carry):
        deg_v[pl.ds(i * 16, 16)] = zeros16
        return carry

    lax.fori_loop(0, N_NODES // 16, zero_body, 0)

    pltpu.sync_copy(dst_hbm.at[wid], dst_v)

    ones16 = jnp.full((16,), 1.0, jnp.float32)

    def body(i, carry):
        idx = dst_v[pl.ds(i * 16, 16)]
        plsc.addupdate_scatter(deg_v, [idx], ones16)
        return carry

    lax.fori_loop(0, EPT // 16, body, 0)

    pltpu.sync_copy(deg_v, out_hbm.at[wid])


@functools.partial(
    pl.kernel,
    out_type=jax.ShapeDtypeStruct((NC, N_NODES, D), jnp.float32),
    mesh=_mesh,
    compiler_params=pltpu.CompilerParams(needs_layout_passes=False),
    scratch_types=[
        pltpu.VMEM((3, CHUNK), jnp.int32),
        pltpu.VMEM((3, CHUNK), jnp.int32),
        pltpu.VMEM((3, CHUNK, D), jnp.float32),
        pltpu.VMEM_SHARED((N_NODES, D), jnp.float32),
        pltpu.SemaphoreType.DMA,
        pltpu.SemaphoreType.DMA,
        pltpu.SemaphoreType.DMA,
        pltpu.SemaphoreType.DMA,
    ],
)
def _edge_kernel(src_hbm, dst_hbm, y_hbm, out_hbm,
                 srci, dsti, rows_v, accum, gsem, isem, jsem, ssem):
    c = lax.axis_index("c")
    s = lax.axis_index("s")
    wid = c * NS + s

    zeros16 = jnp.zeros((16,), jnp.float32)

    def zb(i, carry):
        rows_v[0, i // 8, pl.ds((i % 8) * 16, 16)] = zeros16
        return carry

    lax.fori_loop(0, CHUNK * 8, zb, 0)

    pltpu.async_copy(src_hbm.at[wid, 0], srci.at[0], isem)
    pltpu.async_copy(src_hbm.at[wid, 1], srci.at[1], isem)
    pltpu.async_copy(dst_hbm.at[wid, 0], dsti.at[0], jsem)

    zrows = rows_v.at[0]
    start = BASE_ROWS * s + 8 * jnp.minimum(s, 2)
    for j in range(BASE_ROWS // CHUNK):
        off = pl.multiple_of(start + j * CHUNK, 8)
        pltpu.sync_copy(zrows, accum.at[pl.ds(off, CHUNK)])
    off64 = pl.multiple_of(start + (BASE_ROWS // CHUNK) * CHUNK, 8)
    pltpu.sync_copy(zrows.at[pl.ds(0, BASE_ROWS % CHUNK)],
                    accum.at[pl.ds(off64, BASE_ROWS % CHUNK)])

    @pl.when(s < 2)
    def _zero_extra():
        off = pl.multiple_of(start + BASE_ROWS, 8)
        pltpu.sync_copy(zrows.at[pl.ds(0, 8)], accum.at[pl.ds(off, 8)])

    plsc.subcore_barrier()

    pltpu.make_async_copy(src_hbm.at[wid, 0], srci.at[0], isem).wait()
    pltpu.async_copy(y_hbm.at[srci.at[0]], rows_v.at[0], gsem)

    def body(k, carry):
        b0 = lax.rem(k, 3)
        b1 = lax.rem(k + 1, 3)

        @pl.when(k >= 2)
        def _drain():
            pltpu.make_async_copy(
                rows_v.at[b1], accum.at[dsti.at[b1]], ssem).wait()

        @pl.when(k + 1 < NCHUNK)
        def _next_gather():
            pltpu.make_async_copy(src_hbm.at[wid, k + 1], srci.at[b1],
                                  isem).wait()
            pltpu.async_copy(y_hbm.at[srci.at[b1]], rows_v.at[b1], gsem)
            pltpu.async_copy(dst_hbm.at[wid, k + 1], dsti.at[b1], jsem)

        @pl.when(k + 2 < NCHUNK)
        def _next_srci():
            pltpu.async_copy(src_hbm.at[wid, k + 2], srci.at[lax.rem(k + 2, 3)],
                             isem)

        pltpu.make_async_copy(y_hbm.at[srci.at[b0]], rows_v.at[b0], gsem).wait()
        pltpu.make_async_copy(dst_hbm.at[wid, k], dsti.at[b0], jsem).wait()
        pltpu.async_copy(rows_v.at[b0], accum.at[dsti.at[b0]], ssem, add=True)
        return carry

    lax.fori_loop(0, NCHUNK, body, 0)

    for k in (NCHUNK - 2, NCHUNK - 1):
        b = k % 3
        pltpu.make_async_copy(rows_v.at[b], accum.at[dsti.at[b]], ssem).wait()
    plsc.subcore_barrier()

    off0 = pl.multiple_of(start, 8)
    pltpu.sync_copy(accum.at[pl.ds(off0, BASE_ROWS)],
                    out_hbm.at[c, pl.ds(off0, BASE_ROWS)])

    @pl.when(s < 2)
    def _write_extra():
        off = pl.multiple_of(start + BASE_ROWS, 8)
        pltpu.sync_copy(accum.at[pl.ds(off, 8)],
                        out_hbm.at[c, pl.ds(off, 8)])


def _mm_body(x_ref, w_ref, degp_ref, y_ref):
    deg = jnp.sum(degp_ref[...], axis=1) + 1.0
    dis = lax.rsqrt(deg)
    xw = jnp.dot(x_ref[...], w_ref[...], preferred_element_type=jnp.float32)
    y_ref[...] = xw * dis[:, None]


def _mm_call(x, W, degp):
    return pl.pallas_call(
        _mm_body,
        grid=(GRID,),
        in_specs=[
            pl.BlockSpec((ROWBLK, D), lambda i: (i, 0)),
            pl.BlockSpec((D, D), lambda i: (0, 0)),
            pl.BlockSpec((ROWBLK, NW), lambda i: (i, 0)),
        ],
        out_specs=pl.BlockSpec((ROWBLK, D), lambda i: (i, 0)),
        out_shape=jax.ShapeDtypeStruct((N_NODES, D), jnp.float32),
    )(x, W, degp)


def _fin_body(q_ref, y_ref, degp_ref, b_ref, o_ref):
    deg = jnp.sum(degp_ref[...], axis=1) + 1.0
    dis = lax.rsqrt(deg)
    t = q_ref[0] + q_ref[1] + y_ref[...]
    o_ref[...] = jnp.maximum(t * dis[:, None] + b_ref[...], 0.0)


def _fin_call(q, y, degp, b2):
    return pl.pallas_call(
        _fin_body,
        grid=(GRID,),
        in_specs=[
            pl.BlockSpec((NC, ROWBLK, D), lambda i: (0, i, 0)),
            pl.BlockSpec((ROWBLK, D), lambda i: (i, 0)),
            pl.BlockSpec((ROWBLK, NW), lambda i: (i, 0)),
            pl.BlockSpec((1, D), lambda i: (0, 0)),
        ],
        out_specs=pl.BlockSpec((ROWBLK, D), lambda i: (i, 0)),
        out_shape=jax.ShapeDtypeStruct((N_NODES, D), jnp.float32),
    )(q, y, degp, b2)


def kernel(x, edge_index, W, b):
    src = edge_index[0].astype(jnp.int32)
    dst = edge_index[1].astype(jnp.int32)
    src3 = src.reshape(NW, NCHUNK, CHUNK)
    dst3 = dst.reshape(NW, NCHUNK, CHUNK)
    dst2 = dst.reshape(NW, EPT)

    degp = _deg_kernel(dst2).T
    y = _mm_call(x, W, degp)
    q = _edge_kernel(src3, dst3, y)
    out = _fin_call(q, y, degp, b.reshape(1, D))
    return out


# speedup vs baseline: 47.5878x; 1.0154x over previous
import functools

import jax
import jax.numpy as jnp
from jax import lax
from jax.experimental import pallas as pl
from jax.experimental.pallas import tpu as pltpu
from jax.experimental.pallas import tpu_sc as plsc

N_NODES = 10000
N_EDGES = 320000
D = 128

NC = 2
NS = 16
NW = NC * NS
EPT = N_EDGES // NW
CHUNK = 80
NCHUNK = EPT // CHUNK
BASE_ROWS = 624
ZROWS = 208

ROWBLK = 2000
GRID = N_NODES // ROWBLK

_mesh = plsc.VectorSubcoreMesh(core_axis_name="c", subcore_axis_name="s")


@functools.partial(
    pl.kernel,
    out_type=jax.ShapeDtypeStruct((NW, N_NODES), jnp.float32),
    mesh=_mesh,
    compiler_params=pltpu.CompilerParams(needs_layout_passes=False),
    scratch_types=[
        pltpu.VMEM((EPT,), jnp.int32),
        pltpu.VMEM((N_NODES,), jnp.float32),
    ],
)
def _deg_kernel(dst_hbm, out_hbm, dst_v, deg_v):
    c = lax.axis_index("c")
    s = lax.axis_index("s")
    wid = c * NS + s

    zeros16 = jnp.zeros((16,), jnp.float32)

    def zero_body(i, carry):
        deg_v[pl.ds(i * 16, 16)] = zeros16
        return carry

    lax.fori_loop(0, N_NODES // 16, zero_body, 0)

    pltpu.sync_copy(dst_hbm.at[wid], dst_v)

    ones16 = jnp.full((16,), 1.0, jnp.float32)

    def body(i, carry):
        idx = dst_v[pl.ds(i * 16, 16)]
        plsc.addupdate_scatter(deg_v, [idx], ones16)
        return carry

    lax.fori_loop(0, EPT // 16, body, 0)

    pltpu.sync_copy(deg_v, out_hbm.at[wid])


@functools.partial(
    pl.kernel,
    out_type=jax.ShapeDtypeStruct((NC, N_NODES, D), jnp.float32),
    mesh=_mesh,
    compiler_params=pltpu.CompilerParams(needs_layout_passes=False),
    scratch_types=[
        pltpu.VMEM((3, CHUNK), jnp.int32),
        pltpu.VMEM((3, CHUNK), jnp.int32),
        pltpu.VMEM((3, CHUNK, D), jnp.float32),
        pltpu.VMEM_SHARED((N_NODES, D), jnp.float32),
        pltpu.SemaphoreType.DMA,
        pltpu.SemaphoreType.DMA,
        pltpu.SemaphoreType.DMA,
        pltpu.SemaphoreType.DMA,
    ],
)
def _edge_kernel(src_hbm, dst_hbm, y_hbm, out_hbm,
                 srci, dsti, rows_v, accum, gsem, isem, jsem, ssem):
    c = lax.axis_index("c")
    s = lax.axis_index("s")
    wid = c * NS + s

    zeros16 = jnp.zeros((16,), jnp.float32)

    def zb(i, carry):
        rows_v[0, i // 8, pl.ds((i % 8) * 16, 16)] = zeros16
        return carry

    lax.fori_loop(0, CHUNK * 8, zb, 0)

    pltpu.async_copy(src_hbm.at[wid, 0], srci.at[0], isem)
    pltpu.async_copy(src_hbm.at[wid, 1], srci.at[1], isem)
    pltpu.async_copy(dst_hbm.at[wid, 0], dsti.at[0], jsem)

    zrows = rows_v.at[0]
    start = BASE_ROWS * s + 8 * jnp.minimum(s, 2)
    for j in range(BASE_ROWS // CHUNK):
        off = pl.multiple_of(start + j * CHUNK, 8)
        pltpu.sync_copy(zrows, accum.at[pl.ds(off, CHUNK)])
    off64 = pl.multiple_of(start + (BASE_ROWS // CHUNK) * CHUNK, 8)
    pltpu.sync_copy(zrows.at[pl.ds(0, BASE_ROWS % CHUNK)],
                    accum.at[pl.ds(off64, BASE_ROWS % CHUNK)])

    @pl.when(s < 2)
    def _zero_extra():
        off = pl.multiple_of(start + BASE_ROWS, 8)
        pltpu.sync_copy(zrows.at[pl.ds(0, 8)], accum.at[pl.ds(off, 8)])

    plsc.subcore_barrier()

    pltpu.make_async_copy(src_hbm.at[wid, 0], srci.at[0], isem).wait()
    pltpu.async_copy(y_hbm.at[srci.at[0]], rows_v.at[0], gsem)

    def body(k, carry):
        b0 = lax.rem(k, 3)
        b1 = lax.rem(k + 1, 3)

        @pl.when(k >= 2)
        def _drain():
            pltpu.make_async_copy(
                rows_v.at[b1], accum.at[dsti.at[b1]], ssem).wait()

        @pl.when(k + 1 < NCHUNK)
        def _next_gather():
            pltpu.make_async_copy(src_hbm.at[wid, k + 1], srci.at[b1],
                                  isem).wait()
            pltpu.async_copy(y_hbm.at[srci.at[b1]], rows_v.at[b1], gsem)
            pltpu.async_copy(dst_hbm.at[wid, k + 1], dsti.at[b1], jsem)

        @pl.when(k + 2 < NCHUNK)
        def _next_srci():
            pltpu.async_copy(src_hbm.at[wid, k + 2], srci.at[lax.rem(k + 2, 3)],
                             isem)

        pltpu.make_async_copy(y_hbm.at[srci.at[b0]], rows_v.at[b0], gsem).wait()
        pltpu.make_async_copy(dst_hbm.at[wid, k], dsti.at[b0], jsem).wait()
        pltpu.async_copy(rows_v.at[b0], accum.at[dsti.at[b0]], ssem, add=True)
        return carry

    lax.fori_loop(0, NCHUNK, body, 0)

    for k in (NCHUNK - 2, NCHUNK - 1):
        b = k % 3
        pltpu.make_async_copy(rows_v.at[b], accum.at[dsti.at[b]], ssem).wait()
    plsc.subcore_barrier()

    off0 = pl.multiple_of(start, 8)
    pltpu.sync_copy(accum.at[pl.ds(off0, BASE_ROWS)],
                    out_hbm.at[c, pl.ds(off0, BASE_ROWS)])

    @pl.when(s < 2)
    def _write_extra():
        off = pl.multiple_of(start + BASE_ROWS, 8)
        pltpu.sync_copy(accum.at[pl.ds(off, 8)],
                        out_hbm.at[c, pl.ds(off, 8)])


def _mm_body(x_ref, w_ref, dis_ref, y_ref):
    xw = jnp.dot(x_ref[...], w_ref[...], preferred_element_type=jnp.float32)
    y_ref[...] = xw * dis_ref[...]


def _mm_call(x, W, dis):
    return pl.pallas_call(
        _mm_body,
        grid=(GRID,),
        in_specs=[
            pl.BlockSpec((ROWBLK, D), lambda i: (i, 0)),
            pl.BlockSpec((D, D), lambda i: (0, 0)),
            pl.BlockSpec((ROWBLK, 1), lambda i: (i, 0)),
        ],
        out_specs=pl.BlockSpec((ROWBLK, D), lambda i: (i, 0)),
        out_shape=jax.ShapeDtypeStruct((N_NODES, D), jnp.float32),
    )(x, W, dis)


def _fin_body(q_ref, y_ref, dis_ref, b_ref, o_ref):
    t = q_ref[0] + q_ref[1] + y_ref[...]
    o_ref[...] = jnp.maximum(t * dis_ref[...] + b_ref[...], 0.0)


def _fin_call(q, y, dis, b2):
    return pl.pallas_call(
        _fin_body,
        grid=(GRID,),
        in_specs=[
            pl.BlockSpec((NC, ROWBLK, D), lambda i: (0, i, 0)),
            pl.BlockSpec((ROWBLK, D), lambda i: (i, 0)),
            pl.BlockSpec((ROWBLK, 1), lambda i: (i, 0)),
            pl.BlockSpec((1, D), lambda i: (0, 0)),
        ],
        out_specs=pl.BlockSpec((ROWBLK, D), lambda i: (i, 0)),
        out_shape=jax.ShapeDtypeStruct((N_NODES, D), jnp.float32),
    )(q, y, dis, b2)


def kernel(x, edge_index, W, b):
    src = edge_index[0].astype(jnp.int32)
    dst = edge_index[1].astype(jnp.int32)
    src3 = src.reshape(NW, NCHUNK, CHUNK)
    dst3 = dst.reshape(NW, NCHUNK, CHUNK)
    dst2 = dst.reshape(NW, EPT)

    degp = _deg_kernel(dst2)
    dis = lax.rsqrt(jnp.sum(degp, axis=0) + 1.0).reshape(N_NODES, 1)
    y = _mm_call(x, W, dis)
    q = _edge_kernel(src3, dst3, y)
    out = _fin_call(q, y, dis, b.reshape(1, D))
    return out


# speedup vs baseline: 48.9503x; 1.0286x over previous
import functools

import jax
import jax.numpy as jnp
from jax import lax
from jax.experimental import pallas as pl
from jax.experimental.pallas import tpu as pltpu
from jax.experimental.pallas import tpu_sc as plsc

N_NODES = 10000
N_EDGES = 320000
D = 128

NC = 2
NS = 16
NW = NC * NS
EPT = N_EDGES // NW
CHUNK = 80
NCHUNK = EPT // CHUNK
BASE_ROWS = 624
ZROWS = 160

ROWBLK = 2000
GRID = N_NODES // ROWBLK

_mesh = plsc.VectorSubcoreMesh(core_axis_name="c", subcore_axis_name="s")


@functools.partial(
    pl.kernel,
    out_type=jax.ShapeDtypeStruct((NW, N_NODES), jnp.float32),
    mesh=_mesh,
    compiler_params=pltpu.CompilerParams(needs_layout_passes=False),
    scratch_types=[
        pltpu.VMEM((EPT,), jnp.int32),
        pltpu.VMEM((N_NODES,), jnp.float32),
    ],
)
def _deg_kernel(dst_hbm, out_hbm, dst_v, deg_v):
    c = lax.axis_index("c")
    s = lax.axis_index("s")
    wid = c * NS + s

    zeros16 = jnp.zeros((16,), jnp.float32)

    def zero_body(i, carry):
        deg_v[pl.ds(i * 16, 16)] = zeros16
        return carry

    lax.fori_loop(0, N_NODES // 16, zero_body, 0)

    pltpu.sync_copy(dst_hbm.at[wid], dst_v)

    ones16 = jnp.full((16,), 1.0, jnp.float32)

    def body(i, carry):
        idx = dst_v[pl.ds(i * 16, 16)]
        plsc.addupdate_scatter(deg_v, [idx], ones16)
        return carry

    lax.fori_loop(0, EPT // 16, body, 0)

    pltpu.sync_copy(deg_v, out_hbm.at[wid])


@functools.partial(
    pl.kernel,
    out_type=jax.ShapeDtypeStruct((NC, N_NODES, D), jnp.float32),
    mesh=_mesh,
    compiler_params=pltpu.CompilerParams(needs_layout_passes=False),
    scratch_types=[
        pltpu.VMEM((4, CHUNK), jnp.int32),
        pltpu.VMEM((4, CHUNK), jnp.int32),
        pltpu.VMEM((4, CHUNK, D), jnp.float32),
        pltpu.VMEM_SHARED((N_NODES, D), jnp.float32),
        pltpu.SemaphoreType.DMA,
        pltpu.SemaphoreType.DMA,
        pltpu.SemaphoreType.DMA,
        pltpu.SemaphoreType.DMA,
    ],
)
def _edge_kernel(src_hbm, dst_hbm, y_hbm, out_hbm,
                 srci, dsti, rows_v, accum, gsem, isem, jsem, ssem):
    c = lax.axis_index("c")
    s = lax.axis_index("s")
    wid = c * NS + s

    pltpu.async_copy(src_hbm.at[wid, 0], srci.at[0], isem)
    pltpu.async_copy(src_hbm.at[wid, 1], srci.at[1], isem)
    pltpu.async_copy(src_hbm.at[wid, 2], srci.at[2], isem)
    pltpu.async_copy(dst_hbm.at[wid, 0], dsti.at[0], jsem)
    pltpu.async_copy(dst_hbm.at[wid, 1], dsti.at[1], jsem)

    zeros16 = jnp.zeros((16,), jnp.float32)

    def zb(i, carry):
        rows_v[0, i // 8, pl.ds((i % 8) * 16, 16)] = zeros16
        return carry

    lax.fori_loop(0, CHUNK * 8, zb, 0)

    zrows = rows_v.at[0]
    start = BASE_ROWS * s + 16 * jnp.minimum(s, 1)
    for j in range(BASE_ROWS // CHUNK):
        off = pl.multiple_of(start + j * CHUNK, 8)
        pltpu.sync_copy(zrows, accum.at[pl.ds(off, CHUNK)])
    off64 = pl.multiple_of(start + (BASE_ROWS // CHUNK) * CHUNK, 8)
    pltpu.sync_copy(zrows.at[pl.ds(0, BASE_ROWS % CHUNK)],
                    accum.at[pl.ds(off64, BASE_ROWS % CHUNK)])

    @pl.when(s < 1)
    def _zero_extra():
        off = pl.multiple_of(start + BASE_ROWS, 16)
        pltpu.sync_copy(zrows.at[pl.ds(0, 16)], accum.at[pl.ds(off, 16)])

    plsc.subcore_barrier()

    pltpu.make_async_copy(src_hbm.at[wid, 0], srci.at[0], isem).wait()
    pltpu.async_copy(y_hbm.at[srci.at[0]], rows_v.at[0], gsem)
    pltpu.make_async_copy(src_hbm.at[wid, 1], srci.at[1], isem).wait()
    pltpu.async_copy(y_hbm.at[srci.at[1]], rows_v.at[1], gsem)

    def body(k, carry):
        b0 = lax.rem(k, 4)
        b2 = lax.rem(k + 2, 4)

        @pl.when(k >= 2)
        def _drain():
            pltpu.make_async_copy(
                rows_v.at[b2], accum.at[dsti.at[b2]], ssem).wait()

        @pl.when(k + 2 < NCHUNK)
        def _next_gather():
            pltpu.make_async_copy(src_hbm.at[wid, k + 2], srci.at[b2],
                                  isem).wait()
            pltpu.async_copy(y_hbm.at[srci.at[b2]], rows_v.at[b2], gsem)
            pltpu.async_copy(dst_hbm.at[wid, k + 2], dsti.at[b2], jsem)

        @pl.when(k + 3 < NCHUNK)
        def _next_srci():
            pltpu.async_copy(src_hbm.at[wid, k + 3], srci.at[lax.rem(k + 3, 4)],
                             isem)

        pltpu.make_async_copy(y_hbm.at[srci.at[b0]], rows_v.at[b0], gsem).wait()
        pltpu.make_async_copy(dst_hbm.at[wid, k], dsti.at[b0], jsem).wait()
        pltpu.async_copy(rows_v.at[b0], accum.at[dsti.at[b0]], ssem, add=True)
        return carry

    lax.fori_loop(0, NCHUNK, body, 0)

    for k in (NCHUNK - 2, NCHUNK - 1):
        b = k % 4
        pltpu.make_async_copy(rows_v.at[b], accum.at[dsti.at[b]], ssem).wait()
    plsc.subcore_barrier()

    off0 = pl.multiple_of(start, 16)
    pltpu.sync_copy(accum.at[pl.ds(off0, BASE_ROWS)],
                    out_hbm.at[c, pl.ds(off0, BASE_ROWS)])

    @pl.when(s < 1)
    def _write_extra():
        off = pl.multiple_of(start + BASE_ROWS, 16)
        pltpu.sync_copy(accum.at[pl.ds(off, 16)],
                        out_hbm.at[c, pl.ds(off, 16)])


def _mm_body(x_ref, w_ref, dis_ref, y_ref):
    xw = jnp.dot(x_ref[...], w_ref[...], preferred_element_type=jnp.float32)
    y_ref[...] = xw * dis_ref[...]


def _mm_call(x, W, dis):
    return pl.pallas_call(
        _mm_body,
        grid=(GRID,),
        in_specs=[
            pl.BlockSpec((ROWBLK, D), lambda i: (i, 0)),
            pl.BlockSpec((D, D), lambda i: (0, 0)),
            pl.BlockSpec((ROWBLK, 1), lambda i: (i, 0)),
        ],
        out_specs=pl.BlockSpec((ROWBLK, D), lambda i: (i, 0)),
        out_shape=jax.ShapeDtypeStruct((N_NODES, D), jnp.float32),
    )(x, W, dis)


def _fin_body(q_ref, y_ref, dis_ref, b_ref, o_ref):
    t = q_ref[0] + q_ref[1] + y_ref[...]
    o_ref[...] = jnp.maximum(t * dis_ref[...] + b_ref[...], 0.0)


def _fin_call(q, y, dis, b2):
    return pl.pallas_call(
        _fin_body,
        grid=(GRID,),
        in_specs=[
            pl.BlockSpec((NC, ROWBLK, D), lambda i: (0, i, 0)),
            pl.BlockSpec((ROWBLK, D), lambda i: (i, 0)),
            pl.BlockSpec((ROWBLK, 1), lambda i: (i, 0)),
            pl.BlockSpec((1, D), lambda i: (0, 0)),
        ],
        out_specs=pl.BlockSpec((ROWBLK, D), lambda i: (i, 0)),
        out_shape=jax.ShapeDtypeStruct((N_NODES, D), jnp.float32),
    )(q, y, dis, b2)


def kernel(x, edge_index, W, b):
    src = edge_index[0].astype(jnp.int32)
    dst = edge_index[1].astype(jnp.int32)
    src3 = src.reshape(NW, NCHUNK, CHUNK)
    dst3 = dst.reshape(NW, NCHUNK, CHUNK)
    dst2 = dst.reshape(NW, EPT)

    degp = _deg_kernel(dst2)
    dis = lax.rsqrt(jnp.sum(degp, axis=0) + 1.0).reshape(N_NODES, 1)
    y = _mm_call(x, W, dis)
    q = _edge_kernel(src3, dst3, y)
    out = _fin_call(q, y, dis, b.reshape(1, D))
    return out


# speedup vs baseline: 49.6697x; 1.0147x over previous
import functools

import jax
import jax.numpy as jnp
from jax import lax
from jax.experimental import pallas as pl
from jax.experimental.pallas import tpu as pltpu
from jax.experimental.pallas import tpu_sc as plsc

N_NODES = 10000
N_EDGES = 320000
D = 128

NC = 2
NS = 16
NW = NC * NS
EPT = N_EDGES // NW
CHUNK = 80
NCHUNK = EPT // CHUNK
BASE_ROWS = 624
ZROWS = 160

ROWBLK = 5000
GRID = N_NODES // ROWBLK

_mesh = plsc.VectorSubcoreMesh(core_axis_name="c", subcore_axis_name="s")


@functools.partial(
    pl.kernel,
    out_type=jax.ShapeDtypeStruct((NW, N_NODES), jnp.float32),
    mesh=_mesh,
    compiler_params=pltpu.CompilerParams(needs_layout_passes=False),
    scratch_types=[
        pltpu.VMEM((EPT,), jnp.int32),
        pltpu.VMEM((N_NODES,), jnp.float32),
    ],
)
def _deg_kernel(dst_hbm, out_hbm, dst_v, deg_v):
    c = lax.axis_index("c")
    s = lax.axis_index("s")
    wid = c * NS + s

    zeros16 = jnp.zeros((16,), jnp.float32)

    def zero_body(i, carry):
        deg_v[pl.ds(i * 16, 16)] = zeros16
        return carry

    lax.fori_loop(0, N_NODES // 16, zero_body, 0)

    pltpu.sync_copy(dst_hbm.at[wid], dst_v)

    ones16 = jnp.full((16,), 1.0, jnp.float32)

    def body(i, carry):
        idx = dst_v[pl.ds(i * 16, 16)]
        plsc.addupdate_scatter(deg_v, [idx], ones16)
        return carry

    lax.fori_loop(0, EPT // 16, body, 0)

    pltpu.sync_copy(deg_v, out_hbm.at[wid])


@functools.partial(
    pl.kernel,
    out_type=jax.ShapeDtypeStruct((NC, N_NODES, D), jnp.float32),
    mesh=_mesh,
    compiler_params=pltpu.CompilerParams(needs_layout_passes=False),
    scratch_types=[
        pltpu.VMEM((4, CHUNK), jnp.int32),
        pltpu.VMEM((4, CHUNK), jnp.int32),
        pltpu.VMEM((4, CHUNK, D), jnp.float32),
        pltpu.VMEM_SHARED((N_NODES, D), jnp.float32),
        pltpu.SemaphoreType.DMA,
        pltpu.SemaphoreType.DMA,
        pltpu.SemaphoreType.DMA,
        pltpu.SemaphoreType.DMA,
    ],
)
def _edge_kernel(src_hbm, dst_hbm, y_hbm, out_hbm,
                 srci, dsti, rows_v, accum, gsem, isem, jsem, ssem):
    c = lax.axis_index("c")
    s = lax.axis_index("s")
    wid = c * NS + s

    pltpu.async_copy(src_hbm.at[wid, 0], srci.at[2], isem)
    pltpu.async_copy(src_hbm.at[wid, 1], srci.at[3], isem)
    pltpu.async_copy(src_hbm.at[wid, 2], srci.at[0], isem)
    pltpu.async_copy(dst_hbm.at[wid, 0], dsti.at[2], jsem)
    pltpu.async_copy(dst_hbm.at[wid, 1], dsti.at[3], jsem)

    pltpu.make_async_copy(src_hbm.at[wid, 0], srci.at[2], isem).wait()
    pltpu.async_copy(y_hbm.at[srci.at[2]], rows_v.at[2], gsem)
    pltpu.make_async_copy(src_hbm.at[wid, 1], srci.at[3], isem).wait()
    pltpu.async_copy(y_hbm.at[srci.at[3]], rows_v.at[3], gsem)

    zeros16 = jnp.zeros((16,), jnp.float32)

    def zb(i, carry):
        rows_v[0, i // 8, pl.ds((i % 8) * 16, 16)] = zeros16
        return carry

    lax.fori_loop(0, CHUNK * 8, zb, 0)

    zrows = rows_v.at[0]
    start = BASE_ROWS * s + 16 * jnp.minimum(s, 1)
    for j in range(BASE_ROWS // CHUNK):
        off = pl.multiple_of(start + j * CHUNK, 8)
        pltpu.sync_copy(zrows, accum.at[pl.ds(off, CHUNK)])
    off64 = pl.multiple_of(start + (BASE_ROWS // CHUNK) * CHUNK, 8)
    pltpu.sync_copy(zrows.at[pl.ds(0, BASE_ROWS % CHUNK)],
                    accum.at[pl.ds(off64, BASE_ROWS % CHUNK)])

    @pl.when(s < 1)
    def _zero_extra():
        off = pl.multiple_of(start + BASE_ROWS, 16)
        pltpu.sync_copy(zrows.at[pl.ds(0, 16)], accum.at[pl.ds(off, 16)])

    plsc.subcore_barrier()

    def body(k, carry):
        b0 = lax.rem(k + 2, 4)
        b2 = lax.rem(k, 4)

        @pl.when(k >= 2)
        def _drain():
            pltpu.make_async_copy(
                rows_v.at[b2], accum.at[dsti.at[b2]], ssem).wait()

        @pl.when(k + 2 < NCHUNK)
        def _next_gather():
            pltpu.make_async_copy(src_hbm.at[wid, k + 2], srci.at[b2],
                                  isem).wait()
            pltpu.async_copy(y_hbm.at[srci.at[b2]], rows_v.at[b2], gsem)
            pltpu.async_copy(dst_hbm.at[wid, k + 2], dsti.at[b2], jsem)

        @pl.when(k + 3 < NCHUNK)
        def _next_srci():
            pltpu.async_copy(src_hbm.at[wid, k + 3], srci.at[lax.rem(k + 1, 4)],
                             isem)

        pltpu.make_async_copy(y_hbm.at[srci.at[b0]], rows_v.at[b0], gsem).wait()
        pltpu.make_async_copy(dst_hbm.at[wid, k], dsti.at[b0], jsem).wait()
        pltpu.async_copy(rows_v.at[b0], accum.at[dsti.at[b0]], ssem, add=True)
        return carry

    lax.fori_loop(0, NCHUNK, body, 0)

    for k in (NCHUNK - 2, NCHUNK - 1):
        b = (k + 2) % 4
        pltpu.make_async_copy(rows_v.at[b], accum.at[dsti.at[b]], ssem).wait()
    plsc.subcore_barrier()

    off0 = pl.multiple_of(start, 16)
    pltpu.sync_copy(accum.at[pl.ds(off0, BASE_ROWS)],
                    out_hbm.at[c, pl.ds(off0, BASE_ROWS)])

    @pl.when(s < 1)
    def _write_extra():
        off = pl.multiple_of(start + BASE_ROWS, 16)
        pltpu.sync_copy(accum.at[pl.ds(off, 16)],
                        out_hbm.at[c, pl.ds(off, 16)])


def _xw_body(x_ref, w_ref, xw_ref):
    xw_ref[...] = jnp.dot(x_ref[...], w_ref[...],
                          preferred_element_type=jnp.float32)


def _xw_call(x, W):
    return pl.pallas_call(
        _xw_body,
        grid=(GRID,),
        in_specs=[
            pl.BlockSpec((ROWBLK, D), lambda i: (i, 0)),
            pl.BlockSpec((D, D), lambda i: (0, 0)),
        ],
        out_specs=pl.BlockSpec((ROWBLK, D), lambda i: (i, 0)),
        out_shape=jax.ShapeDtypeStruct((N_NODES, D), jnp.float32),
    )(x, W)


def _scale_body(xw_ref, dis_ref, y_ref):
    y_ref[...] = xw_ref[...] * dis_ref[...]


def _scale_call(xw, dis):
    return pl.pallas_call(
        _scale_body,
        grid=(GRID,),
        in_specs=[
            pl.BlockSpec((ROWBLK, D), lambda i: (i, 0)),
            pl.BlockSpec((ROWBLK, 1), lambda i: (i, 0)),
        ],
        out_specs=pl.BlockSpec((ROWBLK, D), lambda i: (i, 0)),
        out_shape=jax.ShapeDtypeStruct((N_NODES, D), jnp.float32),
    )(xw, dis)


def _fin_body(q_ref, y_ref, dis_ref, b_ref, o_ref):
    t = q_ref[0] + q_ref[1] + y_ref[...]
    o_ref[...] = jnp.maximum(t * dis_ref[...] + b_ref[...], 0.0)


def _fin_call(q, y, dis, b2):
    return pl.pallas_call(
        _fin_body,
        grid=(GRID,),
        in_specs=[
            pl.BlockSpec((NC, ROWBLK, D), lambda i: (0, i, 0)),
            pl.BlockSpec((ROWBLK, D), lambda i: (i, 0)),
            pl.BlockSpec((ROWBLK, 1), lambda i: (i, 0)),
            pl.BlockSpec((1, D), lambda i: (0, 0)),
        ],
        out_specs=pl.BlockSpec((ROWBLK, D), lambda i: (i, 0)),
        out_shape=jax.ShapeDtypeStruct((N_NODES, D), jnp.float32),
    )(q, y, dis, b2)


def kernel(x, edge_index, W, b):
    src = edge_index[0].astype(jnp.int32)
    dst = edge_index[1].astype(jnp.int32)
    src3 = src.reshape(NW, NCHUNK, CHUNK)
    dst3 = dst.reshape(NW, NCHUNK, CHUNK)
    dst2 = dst.reshape(NW, EPT)

    xw = _xw_call(x, W)
    degp = _deg_kernel(dst2)
    dis = lax.rsqrt(jnp.sum(degp, axis=0) + 1.0).reshape(N_NODES, 1)
    y = _scale_call(xw, dis)
    q = _edge_kernel(src3, dst3, y)
    out = _fin_call(q, y, dis, b.reshape(1, D))
    return out
